# baseline (speedup 1.0000x reference)
import functools

import numpy as np
import jax
import jax.numpy as jnp
from jax import lax
from jax.experimental import pallas as pl
from jax.experimental.pallas import tpu as pltpu
from jax.experimental.pallas import tpu_sc as plsc

D_MODEL = 256
N_HEADS = 8
N_LEVELS = 4
N_POINTS = 4
HEAD_DIM = 32
_SHAPES = np.array([[64, 64], [32, 32], [16, 16], [8, 8]], dtype=np.int64)
LEN_IN = int((_SHAPES[:, 0] * _SHAPES[:, 1]).sum())
LQ = LEN_IN
BATCH = 2
ROWS = BATCH * LQ
LP = N_LEVELS * N_POINTS
HLP = N_HEADS * LP
_LEVEL_START = np.concatenate([[0], np.cumsum(_SHAPES[:, 0] * _SHAPES[:, 1])])[:N_LEVELS]

CHUNK = 544
N_BLOCKS = ROWS // CHUNK
BLOCKS_PER_B = LQ // CHUNK

_lv = np.tile(np.repeat(np.arange(N_LEVELS), N_POINTS)[None, :], (N_HEADS, 1)).reshape(-1)
_hv = np.repeat(np.arange(N_HEADS), LP)
_Wc = _SHAPES[_lv, 1].astype(np.float32)
_Hc = _SHAPES[_lv, 0].astype(np.float32)
_GXS = (_Wc / _SHAPES[_lv, 0].astype(np.float32)).reshape(1, HLP)
_GYS = (_Hc / _SHAPES[_lv, 1].astype(np.float32)).reshape(1, HLP)
_WC = _Wc.reshape(1, HLP)
_HC = _Hc.reshape(1, HLP)
_WI = _SHAPES[_lv, 1].astype(np.int32).reshape(1, HLP)
_HI = _SHAPES[_lv, 0].astype(np.int32).reshape(1, HLP)
_COLBASE = (_hv * LEN_IN + _LEVEL_START[_lv]).astype(np.int32).reshape(1, HLP)
_G = (np.arange(HLP)[:, None] // LP == np.arange(HLP)[None, :] // LP).astype(np.float32)
_pv = np.tile(np.arange(N_POINTS), N_HEADS * N_LEVELS)
_perm_x = _hv * (LP * 2) + _lv * (N_POINTS * 2) + _pv * 2
_PERM = np.concatenate([_perm_x, _perm_x + 1])
_OUTPERM = np.empty((D_MODEL,), np.int32)
for _h in range(N_HEADS):
    for _k in range(32):
        _OUTPERM[_h * 32 + _k] = _h * 32 + (2 * _k if _k < 16 else 2 * (_k - 16) + 1)


def _stage_a_body(x_ref, q_ref, rpx_ref, rpy_ref, wv_ref, bv_ref, wo_ref, bo_ref,
                  wa_ref, ba_ref, g_ref, wc_ref, hc_ref, gxs_ref, gys_ref,
                  wi_ref, hi_ref, cb_ref, vout_ref, idx_ref, wgt_ref):
    vout_ref[...] = (jnp.dot(x_ref[...], wv_ref[...],
                             preferred_element_type=jnp.float32)
                     + bv_ref[...]).astype(jnp.bfloat16)
    q = q_ref[...]
    off = jnp.dot(q, wo_ref[...], preferred_element_type=jnp.float32) + bo_ref[...]
    gx = off[:, :HLP]
    gy = off[:, HLP:]
    logits = jnp.dot(q, wa_ref[...], preferred_element_type=jnp.float32) + ba_ref[...]
    m = jnp.max(logits, axis=-1, keepdims=True)
    e = jnp.exp(logits - m)
    s = jnp.dot(e, g_ref[...], preferred_element_type=jnp.float32)
    attn = e / s

    x = rpx_ref[...] * wc_ref[...] + gx * gxs_ref[...] - 0.5
    y = rpy_ref[...] * hc_ref[...] + gy * gys_ref[...] - 0.5
    x0f = jnp.floor(x)
    y0f = jnp.floor(y)
    wx1 = x - x0f
    wx0 = 1.0 - wx1
    wy1 = y - y0f
    wy0 = 1.0 - wy1
    x0 = x0f.astype(jnp.int32)
    y0 = y0f.astype(jnp.int32)
    Wi = wi_ref[...]
    Hi = hi_ref[...]
    vx0 = ((x0 >= 0) & (x0 < Wi)).astype(jnp.float32)
    vx1 = ((x0 + 1 >= 0) & (x0 + 1 < Wi)).astype(jnp.float32)
    vy0 = ((y0 >= 0) & (y0 < Hi)).astype(jnp.float32)
    vy1 = ((y0 + 1 >= 0) & (y0 + 1 < Hi)).astype(jnp.float32)
    xb = jnp.clip(x0, 0, Wi - 2)
    wq0 = wx0 * vx0
    wq1 = wx1 * vx1
    e0 = xb == x0
    ws0 = jnp.where(e0, wq0, wq1)
    ws1 = jnp.where(e0, wq1, jnp.where(xb + 1 == x0, wq0, 0.0))
    y0c = jnp.clip(y0, 0, Hi - 1)
    y1c = jnp.clip(y0 + 1, 0, Hi - 1)
    ay0 = attn * (wy0 * vy0)
    ay1 = attn * (wy1 * vy1)
    b = pl.program_id(0) // BLOCKS_PER_B
    base = cb_ref[...] + b * (N_HEADS * LEN_IN) + xb
    idx_ref[...] = jnp.concatenate([base + y0c * Wi, base + y1c * Wi], axis=1)
    wgt_ref[...] = jnp.concatenate([ay0 * ws0, ay1 * ws0, ay0 * ws1, ay1 * ws1], axis=1)


def _stage_a(x, q, rpx, rpy, W_value, b_value, W_offp, b_offp, W_attn, b_attn):
    full = lambda shp: pl.BlockSpec(shp, lambda i: (0, 0))
    row_blk = lambda w: pl.BlockSpec((CHUNK, w), lambda i: (i, 0))
    consts = dict(
        g=_G, wc=_WC, hc=_HC, gxs=_GXS, gys=_GYS, wi=_WI, hi=_HI, cb=_COLBASE)
    return pl.pallas_call(
        _stage_a_body,
        grid=(N_BLOCKS,),
        in_specs=[
            row_blk(D_MODEL), row_blk(D_MODEL), row_blk(HLP), row_blk(HLP),
            full((D_MODEL, D_MODEL)), full((1, D_MODEL)),
            full((D_MODEL, D_MODEL)), full((1, D_MODEL)),
            full((D_MODEL, HLP)), full((1, HLP)),
            full((HLP, HLP)), full((1, HLP)), full((1, HLP)), full((1, HLP)),
            full((1, HLP)), full((1, HLP)), full((1, HLP)), full((1, HLP)),
        ],
        out_specs=[row_blk(D_MODEL), row_blk(D_MODEL), row_blk(2 * D_MODEL)],
        out_shape=[
            jax.ShapeDtypeStruct((ROWS, D_MODEL), jnp.bfloat16),
            jax.ShapeDtypeStruct((ROWS, 2 * HLP), jnp.int32),
            jax.ShapeDtypeStruct((ROWS, 4 * HLP), jnp.float32),
        ],
    )(x, q, rpx, rpy, W_value, b_value.reshape(1, -1), W_offp,
      b_offp.reshape(1, -1), W_attn, b_attn.reshape(1, -1),
      jnp.asarray(_G), jnp.asarray(_WC), jnp.asarray(_HC), jnp.asarray(_GXS),
      jnp.asarray(_GYS), jnp.asarray(_WI), jnp.asarray(_HI), jnp.asarray(_COLBASE))


QPW = ROWS // 32
QC = 5
NCHUNK = QPW // QC


NSTEP = NCHUNK // 2


def _sc_body(table_hbm, idx_hbm, wgt_hbm, out_hbm,
             idx_v0, idx_v1, wgt_v0, wgt_v1, rows_v0, rows_v1, out_v0, out_v1,
             gsem0, gsem1, msem0, msem1, osem0, osem1):
    cid = lax.axis_index("c")
    sid = lax.axis_index("s")
    wid = sid * 2 + cid
    q0w = wid * QPW
    bufs = ((idx_v0, wgt_v0, rows_v0, out_v0, gsem0, msem0, osem0),
            (idx_v1, wgt_v1, rows_v1, out_v1, gsem1, msem1, osem1))

    def fire_meta(t, b):
        idx_v, wgt_v, _, _, _, msem, _ = bufs[b]
        q0 = q0w + t * QC
        pltpu.async_copy(idx_hbm.at[pl.ds(q0 * 2, QC * 2)], idx_v, msem)
        pltpu.async_copy(wgt_hbm.at[pl.ds(q0 * 512, QC * 512)], wgt_v, msem)

    def wait_meta(b):
        idx_v, wgt_v, _, _, _, msem, _ = bufs[b]
        pltpu.make_async_copy(idx_hbm.at[pl.ds(0, QC * 2)], idx_v, msem).wait()
        pltpu.make_async_copy(wgt_hbm.at[pl.ds(0, QC * 512)], wgt_v, msem).wait()

    def fire_gathers(b):
        idx_v, _, rows_v, _, gsem, _, _ = bufs[b]
        for g in range(2 * QC):
            pltpu.async_copy(table_hbm.at[idx_v.at[g]],
                             rows_v.at[pl.ds(g * 128, 128)], gsem)

    def wait_gathers(b):
        idx_v, _, rows_v, _, gsem, _, _ = bufs[b]
        for g in range(2 * QC):
            pltpu.make_async_copy(table_hbm.at[idx_v.at[g]],
                                  rows_v.at[pl.ds(g * 128, 128)], gsem).wait()

    def fire_out(t, b):
        _, _, _, out_v, _, _, osem = bufs[b]
        q0 = q0w + t * QC
        pltpu.async_copy(out_v, out_hbm.at[pl.ds(q0 * 256, QC * 256)], osem)

    def wait_out(b):
        _, _, _, out_v, _, _, osem = bufs[b]
        pltpu.make_async_copy(out_v, out_hbm.at[pl.ds(0, QC * 256)], osem).wait()

    def compute(b):
        _, wgt, rows, out, _, _, _ = bufs[b]

        def qh_body(i, _):
            q = i // N_HEADS
            h = i - q * N_HEADS
            base_w = q * 512 + h * 16
            base_j = q * 256 + h * 16
            a0 = jnp.zeros((16,), jnp.float32)
            a1 = jnp.zeros((16,), jnp.float32)
            for c in range(2):
                wv_s0 = wgt[pl.ds(base_w + c * 128, 16)]
                wv_s1 = wgt[pl.ds(base_w + 256 + c * 128, 16)]
                for lp in range(LP):
                    j = base_j + c * 128 + lp
                    w0 = jnp.full((16,), wv_s0[lp], jnp.float32)
                    w1 = jnp.full((16,), wv_s1[lp], jnp.float32)
                    e0, o0 = plsc.unpack(rows[j, pl.ds(0, 32)],
                                         format=plsc.PackFormat.INTERLEAVED,
                                         preferred_element_type=jnp.float32)
                    e1, o1 = plsc.unpack(rows[j, pl.ds(32, 32)],
                                         format=plsc.PackFormat.INTERLEAVED,
                                         preferred_element_type=jnp.float32)
                    a0 = a0 + e0 * w0 + e1 * w1
                    a1 = a1 + o0 * w0 + o1 * w1
            out[pl.ds(q * 256 + h * 32, 16)] = a0
            out[pl.ds(q * 256 + h * 32 + 16, 16)] = a1
            return 0

        lax.fori_loop(0, QC * N_HEADS, qh_body, 0)

    fire_meta(0, 0)
    wait_meta(0)
    fire_gathers(0)
    fire_meta(1, 1)

    def step(s, _):
        c0 = 2 * s
        c1 = 2 * s + 1
        not_last = s < NSTEP - 1
        not_first = s > 0
        wait_meta(1)
        fire_gathers(1)
        wait_gathers(0)
        pl.when(not_first)(lambda: wait_out(0))
        compute(0)
        fire_out(c0, 0)
        pl.when(not_last)(lambda: fire_meta(c0 + 2, 0))
        wait_gathers(1)
        pl.when(not_first)(lambda: wait_out(1))
        compute(1)
        fire_out(c1, 1)
        pl.when(not_last)(lambda: fire_meta(c1 + 2, 1))

        def refill():
            wait_meta(0)
            fire_gathers(0)
        pl.when(not_last)(refill)
        return 0

    lax.fori_loop(0, NSTEP, step, 0)
    wait_out(0)
    wait_out(1)


def _sc_weighted_gather(table, idx2, wgt):
    mesh = plsc.VectorSubcoreMesh(core_axis_name="c", subcore_axis_name="s")
    buf_types = [
        pltpu.VMEM((2 * QC, 128), jnp.int32), pltpu.VMEM((2 * QC, 128), jnp.int32),
        pltpu.VMEM((QC * 4 * HLP,), jnp.float32), pltpu.VMEM((QC * 4 * HLP,), jnp.float32),
        pltpu.VMEM((QC * 256, 64), jnp.bfloat16), pltpu.VMEM((QC * 256, 64), jnp.bfloat16),
        pltpu.VMEM((QC * D_MODEL,), jnp.float32), pltpu.VMEM((QC * D_MODEL,), jnp.float32),
    ] + [pltpu.SemaphoreType.DMA] * 6
    fn = functools.partial(
        pl.kernel,
        out_type=jax.ShapeDtypeStruct((ROWS * D_MODEL,), jnp.float32),
        mesh=mesh,
        scratch_types=buf_types,
        compiler_params=pltpu.CompilerParams(use_tc_tiling_on_sc=False,
                                             needs_layout_passes=False),
    )(_sc_body)
    return fn(table, idx2, wgt).reshape(ROWS, D_MODEL)


def _matmul_body(x_ref, w_ref, b_ref, o_ref):
    o_ref[...] = jnp.dot(x_ref[...], w_ref[...],
                         preferred_element_type=jnp.float32) + b_ref[...]


def _stage_c(x, W_out, b_out):
    return pl.pallas_call(
        _matmul_body,
        grid=(N_BLOCKS,),
        in_specs=[
            pl.BlockSpec((CHUNK, D_MODEL), lambda i: (i, 0)),
            pl.BlockSpec((D_MODEL, D_MODEL), lambda i: (0, 0)),
            pl.BlockSpec((1, D_MODEL), lambda i: (0, 0)),
        ],
        out_specs=pl.BlockSpec((CHUNK, D_MODEL), lambda i: (i, 0)),
        out_shape=jax.ShapeDtypeStruct((ROWS, D_MODEL), jnp.float32),
    )(x, W_out, b_out.reshape(1, -1))


def kernel(query, reference_points, input_flatten, input_spatial_shapes,
           W_value, b_value, W_off, b_off, W_attn, b_attn, W_out, b_out):
    x = input_flatten.reshape(ROWS, D_MODEL)
    q = query.reshape(ROWS, D_MODEL)
    rp = reference_points.reshape(ROWS, N_LEVELS, 2)
    rpx = jnp.broadcast_to(rp[:, None, :, None, 0],
                           (ROWS, N_HEADS, N_LEVELS, N_POINTS)).reshape(ROWS, HLP)
    rpy = jnp.broadcast_to(rp[:, None, :, None, 1],
                           (ROWS, N_HEADS, N_LEVELS, N_POINTS)).reshape(ROWS, HLP)
    W_offp = W_off[:, jnp.asarray(_PERM)]
    b_offp = b_off[jnp.asarray(_PERM)]

    value, idx, wgt = _stage_a(x, q, rpx, rpy, W_value, b_value, W_offp, b_offp,
                               W_attn, b_attn)

    v4 = value.reshape(BATCH, LEN_IN, N_HEADS, HEAD_DIM).transpose(0, 2, 1, 3)
    vnext = jnp.concatenate(
        [v4[:, :, 1:], jnp.zeros((BATCH, N_HEADS, 1, HEAD_DIM), jnp.bfloat16)], axis=2)
    table = jnp.concatenate([v4, vnext], axis=-1).reshape(
        BATCH * N_HEADS * LEN_IN, 2 * HEAD_DIM)

    sc_out = _sc_weighted_gather(table, idx.reshape(ROWS * 2, HLP),
                                 wgt.reshape(ROWS * 4 * HLP))
    out = _stage_c(sc_out, W_out[jnp.asarray(_OUTPERM), :], b_out)
    return out.reshape(BATCH, LQ, D_MODEL)


# speedup vs baseline: 1.0800x; 1.0800x over previous
import functools

import numpy as np
import jax
import jax.numpy as jnp
from jax import lax
from jax.experimental import pallas as pl
from jax.experimental.pallas import tpu as pltpu
from jax.experimental.pallas import tpu_sc as plsc

D_MODEL = 256
N_HEADS = 8
N_LEVELS = 4
N_POINTS = 4
HEAD_DIM = 32
_SHAPES = np.array([[64, 64], [32, 32], [16, 16], [8, 8]], dtype=np.int64)
LEN_IN = int((_SHAPES[:, 0] * _SHAPES[:, 1]).sum())
LQ = LEN_IN
BATCH = 2
ROWS = BATCH * LQ
LP = N_LEVELS * N_POINTS
HLP = N_HEADS * LP
_LEVEL_START = np.concatenate([[0], np.cumsum(_SHAPES[:, 0] * _SHAPES[:, 1])])[:N_LEVELS]

CHUNK = 544
N_BLOCKS = ROWS // CHUNK
BLOCKS_PER_B = LQ // CHUNK

_lv = np.tile(np.repeat(np.arange(N_LEVELS), N_POINTS)[None, :], (N_HEADS, 1)).reshape(-1)
_hv = np.repeat(np.arange(N_HEADS), LP)
_Wc = _SHAPES[_lv, 1].astype(np.float32)
_Hc = _SHAPES[_lv, 0].astype(np.float32)
_GXS = (_Wc / _SHAPES[_lv, 0].astype(np.float32)).reshape(1, HLP)
_GYS = (_Hc / _SHAPES[_lv, 1].astype(np.float32)).reshape(1, HLP)
_WC = _Wc.reshape(1, HLP)
_HC = _Hc.reshape(1, HLP)
_WI = _SHAPES[_lv, 1].astype(np.int32).reshape(1, HLP)
_HI = _SHAPES[_lv, 0].astype(np.int32).reshape(1, HLP)
_COLBASE = (_hv * LEN_IN + _LEVEL_START[_lv]).astype(np.int32).reshape(1, HLP)
_G = (np.arange(HLP)[:, None] // LP == np.arange(HLP)[None, :] // LP).astype(np.float32)
_S8 = np.zeros((2 * N_LEVELS, 2 * HLP), np.float32)
for _j in range(HLP):
    _S8[2 * _lv[_j] + 0, _j] = 1.0
    _S8[2 * _lv[_j] + 1, HLP + _j] = 1.0
_pv = np.tile(np.arange(N_POINTS), N_HEADS * N_LEVELS)
_perm_x = _hv * (LP * 2) + _lv * (N_POINTS * 2) + _pv * 2
_PERM = np.concatenate([_perm_x, _perm_x + 1])
_OUTPERM = np.empty((D_MODEL,), np.int32)
for _h in range(N_HEADS):
    for _k in range(32):
        _OUTPERM[_h * 32 + _k] = _h * 32 + (2 * _k if _k < 16 else 2 * (_k - 16) + 1)


def _stage_a_body(x_ref, q_ref, rp_ref, wv_ref, bv_ref, wo_ref, bo_ref,
                  wa_ref, ba_ref, g_ref, s8_ref, wc_ref, hc_ref, gxs_ref, gys_ref,
                  wi_ref, hi_ref, cb_ref, vout_ref, idx0_ref, idx1_ref,
                  w00_ref, w01_ref, w10_ref, w11_ref):
    vout_ref[...] = (jnp.dot(x_ref[...], wv_ref[...],
                             preferred_element_type=jnp.float32)
                     + bv_ref[...]).astype(jnp.bfloat16)
    q = q_ref[...]
    off = jnp.dot(q, wo_ref[...], preferred_element_type=jnp.float32) + bo_ref[...]
    gx = off[:, :HLP]
    gy = off[:, HLP:]
    logits = jnp.dot(q, wa_ref[...], preferred_element_type=jnp.float32) + ba_ref[...]
    m = jnp.max(logits, axis=-1, keepdims=True)
    e = jnp.exp(logits - m)
    s = jnp.dot(e, g_ref[...], preferred_element_type=jnp.float32)
    attn = e / s
    rpxy = jnp.dot(rp_ref[...], s8_ref[...], preferred_element_type=jnp.float32)

    x = rpxy[:, :HLP] * wc_ref[...] + gx * gxs_ref[...] - 0.5
    y = rpxy[:, HLP:] * hc_ref[...] + gy * gys_ref[...] - 0.5
    x0f = jnp.floor(x)
    y0f = jnp.floor(y)
    wx1 = x - x0f
    wx0 = 1.0 - wx1
    wy1 = y - y0f
    wy0 = 1.0 - wy1
    x0 = x0f.astype(jnp.int32)
    y0 = y0f.astype(jnp.int32)
    Wi = wi_ref[...]
    Hi = hi_ref[...]
    vx0 = ((x0 >= 0) & (x0 < Wi)).astype(jnp.float32)
    vx1 = ((x0 + 1 >= 0) & (x0 + 1 < Wi)).astype(jnp.float32)
    vy0 = ((y0 >= 0) & (y0 < Hi)).astype(jnp.float32)
    vy1 = ((y0 + 1 >= 0) & (y0 + 1 < Hi)).astype(jnp.float32)
    xb = jnp.clip(x0, 0, Wi - 2)
    wq0 = wx0 * vx0
    wq1 = wx1 * vx1
    e0 = xb == x0
    ws0 = jnp.where(e0, wq0, wq1)
    ws1 = jnp.where(e0, wq1, jnp.where(xb + 1 == x0, wq0, 0.0))
    y0c = jnp.clip(y0, 0, Hi - 1)
    y1c = jnp.clip(y0 + 1, 0, Hi - 1)
    ay0 = attn * (wy0 * vy0)
    ay1 = attn * (wy1 * vy1)
    b = pl.program_id(0) // BLOCKS_PER_B
    base = cb_ref[...] + b * (N_HEADS * LEN_IN) + xb
    idx0_ref[...] = base + y0c * Wi
    idx1_ref[...] = base + y1c * Wi
    w00_ref[...] = ay0 * ws0
    w01_ref[...] = ay1 * ws0
    w10_ref[...] = ay0 * ws1
    w11_ref[...] = ay1 * ws1


def _stage_a(x, q, rp8, W_value, b_value, W_offp, b_offp, W_attn, b_attn):
    full = lambda shp: pl.BlockSpec(shp, lambda i: (0, 0))
    row_blk = lambda w: pl.BlockSpec((CHUNK, w), lambda i: (i, 0))
    return pl.pallas_call(
        _stage_a_body,
        grid=(N_BLOCKS,),
        in_specs=[
            row_blk(D_MODEL), row_blk(D_MODEL), row_blk(2 * N_LEVELS),
            full((D_MODEL, D_MODEL)), full((1, D_MODEL)),
            full((D_MODEL, D_MODEL)), full((1, D_MODEL)),
            full((D_MODEL, HLP)), full((1, HLP)),
            full((HLP, HLP)), full((2 * N_LEVELS, 2 * HLP)),
            full((1, HLP)), full((1, HLP)), full((1, HLP)),
            full((1, HLP)), full((1, HLP)), full((1, HLP)), full((1, HLP)),
        ],
        out_specs=[row_blk(D_MODEL)] + [row_blk(HLP)] * 6,
        out_shape=[jax.ShapeDtypeStruct((ROWS, D_MODEL), jnp.bfloat16)]
        + [jax.ShapeDtypeStruct((ROWS, HLP), jnp.int32)] * 2
        + [jax.ShapeDtypeStruct((ROWS, HLP), jnp.float32)] * 4,
    )(x, q, rp8, W_value, b_value.reshape(1, -1), W_offp,
      b_offp.reshape(1, -1), W_attn, b_attn.reshape(1, -1),
      jnp.asarray(_G), jnp.asarray(_S8), jnp.asarray(_WC), jnp.asarray(_HC),
      jnp.asarray(_GXS), jnp.asarray(_GYS), jnp.asarray(_WI), jnp.asarray(_HI),
      jnp.asarray(_COLBASE))


QPW = ROWS // 32
QC = 5
NCHUNK = QPW // QC


NSTEP = NCHUNK // 2


def _sc_body(table_hbm, idx0_hbm, idx1_hbm, w00_hbm, w01_hbm, w10_hbm, w11_hbm,
             out_hbm,
             idx_v0, idx_v1, wgt_v0, wgt_v1, rows_v0, rows_v1, out_v0, out_v1,
             gsem0, gsem1, msem0, msem1, osem0, osem1):
    cid = lax.axis_index("c")
    sid = lax.axis_index("s")
    wid = sid * 2 + cid
    q0w = wid * QPW
    bufs = ((idx_v0, wgt_v0, rows_v0, out_v0, gsem0, msem0, osem0),
            (idx_v1, wgt_v1, rows_v1, out_v1, gsem1, msem1, osem1))
    whbm = (w00_hbm, w01_hbm, w10_hbm, w11_hbm)

    def fire_meta(t, b):
        idx_v, wgt_v, _, _, _, msem, _ = bufs[b]
        q0 = q0w + t * QC
        pltpu.async_copy(idx0_hbm.at[pl.ds(q0, QC)], idx_v.at[pl.ds(0, QC)], msem)
        pltpu.async_copy(idx1_hbm.at[pl.ds(q0, QC)], idx_v.at[pl.ds(QC, QC)], msem)
        for k in range(4):
            pltpu.async_copy(whbm[k].at[pl.ds(q0, QC)],
                             wgt_v.at[pl.ds(k * QC, QC)], msem)

    def wait_meta(b):
        idx_v, wgt_v, _, _, _, msem, _ = bufs[b]
        pltpu.make_async_copy(idx0_hbm.at[pl.ds(0, QC)],
                              idx_v.at[pl.ds(0, QC)], msem).wait()
        pltpu.make_async_copy(idx1_hbm.at[pl.ds(0, QC)],
                              idx_v.at[pl.ds(QC, QC)], msem).wait()
        for k in range(4):
            pltpu.make_async_copy(whbm[k].at[pl.ds(0, QC)],
                                  wgt_v.at[pl.ds(k * QC, QC)], msem).wait()

    def fire_gathers(b):
        idx_v, _, rows_v, _, gsem, _, _ = bufs[b]
        for g in range(2 * QC):
            pltpu.async_copy(table_hbm.at[idx_v.at[g]],
                             rows_v.at[pl.ds(g * 128, 128)], gsem)

    def wait_gathers(b):
        idx_v, _, rows_v, _, gsem, _, _ = bufs[b]
        for g in range(2 * QC):
            pltpu.make_async_copy(table_hbm.at[idx_v.at[g]],
                                  rows_v.at[pl.ds(g * 128, 128)], gsem).wait()

    def fire_out(t, b):
        _, _, _, out_v, _, _, osem = bufs[b]
        q0 = q0w + t * QC
        pltpu.async_copy(out_v, out_hbm.at[pl.ds(q0 * 2, QC * 2)], osem)

    def wait_out(b):
        _, _, _, out_v, _, _, osem = bufs[b]
        pltpu.make_async_copy(out_v, out_hbm.at[pl.ds(0, QC * 2)], osem).wait()

    def compute(b):
        _, wgt, rows, out, _, _, _ = bufs[b]

        def qh_body(i, _):
            q = i // N_HEADS
            h = i - q * N_HEADS
            a0 = jnp.zeros((16,), jnp.float32)
            a1 = jnp.zeros((16,), jnp.float32)
            for c in range(2):
                wv_s0 = wgt[c * QC + q, pl.ds(h * 16, 16)]
                wv_s1 = wgt[(2 + c) * QC + q, pl.ds(h * 16, 16)]
                base_j = (c * QC + q) * 128 + h * 16
                for lp in range(LP):
                    j = base_j + lp
                    w0 = jnp.full((16,), wv_s0[lp], jnp.float32)
                    w1 = jnp.full((16,), wv_s1[lp], jnp.float32)
                    e0, o0 = plsc.unpack(rows[j, pl.ds(0, 32)],
                                         format=plsc.PackFormat.INTERLEAVED,
                                         preferred_element_type=jnp.float32)
                    e1, o1 = plsc.unpack(rows[j, pl.ds(32, 32)],
                                         format=plsc.PackFormat.INTERLEAVED,
                                         preferred_element_type=jnp.float32)
                    a0 = a0 + e0 * w0 + e1 * w1
                    a1 = a1 + o0 * w0 + o1 * w1
            out[q * 2 + h // 4, pl.ds((h % 4) * 32, 16)] = a0
            out[q * 2 + h // 4, pl.ds((h % 4) * 32 + 16, 16)] = a1
            return 0

        lax.fori_loop(0, QC * N_HEADS, qh_body, 0)

    fire_meta(0, 0)
    wait_meta(0)
    fire_gathers(0)
    fire_meta(1, 1)

    def step(s, _):
        c0 = 2 * s
        c1 = 2 * s + 1
        not_last = s < NSTEP - 1
        not_first = s > 0
        wait_meta(1)
        fire_gathers(1)
        wait_gathers(0)
        pl.when(not_first)(lambda: wait_out(0))
        compute(0)
        fire_out(c0, 0)
        pl.when(not_last)(lambda: fire_meta(c0 + 2, 0))
        wait_gathers(1)
        pl.when(not_first)(lambda: wait_out(1))
        compute(1)
        fire_out(c1, 1)
        pl.when(not_last)(lambda: fire_meta(c1 + 2, 1))

        def refill():
            wait_meta(0)
            fire_gathers(0)
        pl.when(not_last)(refill)
        return 0

    lax.fori_loop(0, NSTEP, step, 0)
    wait_out(0)
    wait_out(1)


def _sc_weighted_gather(table, idx0, idx1, w00, w01, w10, w11):
    mesh = plsc.VectorSubcoreMesh(core_axis_name="c", subcore_axis_name="s")
    buf_types = [
        pltpu.VMEM((2 * QC, 128), jnp.int32), pltpu.VMEM((2 * QC, 128), jnp.int32),
        pltpu.VMEM((4 * QC, 128), jnp.float32), pltpu.VMEM((4 * QC, 128), jnp.float32),
        pltpu.VMEM((QC * 256, 64), jnp.bfloat16), pltpu.VMEM((QC * 256, 64), jnp.bfloat16),
        pltpu.VMEM((QC * 2, 128), jnp.float32), pltpu.VMEM((QC * 2, 128), jnp.float32),
    ] + [pltpu.SemaphoreType.DMA] * 6
    fn = functools.partial(
        pl.kernel,
        out_type=jax.ShapeDtypeStruct((ROWS * 2, 128), jnp.float32),
        mesh=mesh,
        scratch_types=buf_types,
        compiler_params=pltpu.CompilerParams(use_tc_tiling_on_sc=False,
                                             needs_layout_passes=False),
    )(_sc_body)
    return fn(table, idx0, idx1, w00, w01, w10, w11)


def _matmul_body(x_ref, w_ref, b_ref, o_ref):
    x = x_ref[...].reshape(CHUNK, 2, 128)
    w = w_ref[...]
    o_ref[...] = (jnp.dot(x[:, 0, :], w[:128],
                          preferred_element_type=jnp.float32)
                  + jnp.dot(x[:, 1, :], w[128:],
                            preferred_element_type=jnp.float32) + b_ref[...])


def _stage_c(x, W_out, b_out):
    return pl.pallas_call(
        _matmul_body,
        grid=(N_BLOCKS,),
        in_specs=[
            pl.BlockSpec((2 * CHUNK, 128), lambda i: (i, 0)),
            pl.BlockSpec((D_MODEL, D_MODEL), lambda i: (0, 0)),
            pl.BlockSpec((1, D_MODEL), lambda i: (0, 0)),
        ],
        out_specs=pl.BlockSpec((CHUNK, D_MODEL), lambda i: (i, 0)),
        out_shape=jax.ShapeDtypeStruct((ROWS, D_MODEL), jnp.float32),
    )(x, W_out, b_out.reshape(1, -1))


def kernel(query, reference_points, input_flatten, input_spatial_shapes,
           W_value, b_value, W_off, b_off, W_attn, b_attn, W_out, b_out):
    x = input_flatten.reshape(ROWS, D_MODEL)
    q = query.reshape(ROWS, D_MODEL)
    rp8 = reference_points.reshape(ROWS, 2 * N_LEVELS)
    W_offp = W_off[:, jnp.asarray(_PERM)]
    b_offp = b_off[jnp.asarray(_PERM)]

    value, idx0, idx1, w00, w01, w10, w11 = _stage_a(
        x, q, rp8, W_value, b_value, W_offp, b_offp, W_attn, b_attn)

    v4 = value.reshape(BATCH, LEN_IN, N_HEADS, HEAD_DIM).transpose(0, 2, 1, 3)
    vnext = jnp.concatenate(
        [v4[:, :, 1:], jnp.zeros((BATCH, N_HEADS, 1, HEAD_DIM), jnp.bfloat16)], axis=2)
    table = jnp.concatenate([v4, vnext], axis=-1).reshape(
        BATCH * N_HEADS * LEN_IN, 2 * HEAD_DIM)

    sc_out = _sc_weighted_gather(table, idx0, idx1, w00, w01, w10, w11)
    out = _stage_c(sc_out, W_out[jnp.asarray(_OUTPERM), :], b_out)
    return out.reshape(BATCH, LQ, D_MODEL)
